# baseline (speedup 1.0000x reference)
import functools

import jax
import jax.numpy as jnp
from jax import lax
from jax.experimental import pallas as pl
from jax.experimental.pallas import tpu as pltpu
from jax.experimental.pallas import tpu_sc as plsc

_EPS = 1e-5
_GATHER_CHUNK = 16
_NBUF = 3


def _sc_gather(word_table, idx):
    n = idx.shape[0]
    h = word_table.shape[1]
    mesh = plsc.VectorSubcoreMesh(core_axis_name="core", subcore_axis_name="subcore")
    num_workers = mesh.num_cores * mesh.num_subcores
    b_per_w = n // num_workers
    ch = _GATHER_CHUNK
    nbuf = _NBUF
    nchunks = b_per_w // ch

    @functools.partial(
        pl.kernel,
        out_type=jax.ShapeDtypeStruct((n, h), jnp.float32),
        mesh=mesh,
        scratch_types=[pltpu.VMEM((b_per_w,), jnp.int32)]
        + [pltpu.VMEM((ch, h), jnp.float32) for _ in range(nbuf)]
        + [pltpu.SemaphoreType.DMA for _ in range(2 * nbuf)],
    )
    def gather_kernel(x_hbm, i_hbm, o_hbm, idx_v, *scratch):
        bufs = scratch[:nbuf]
        gsem = scratch[nbuf:2 * nbuf]
        ssem = scratch[2 * nbuf:]
        wid = lax.axis_index("subcore") * mesh.num_cores + lax.axis_index("core")
        base = wid * b_per_w
        pltpu.sync_copy(i_hbm.at[pl.ds(base, b_per_w)], idx_v)

        g_h = [None] * nchunks
        s_h = [None] * nchunks

        def start_g(c):
            g_h[c] = pltpu.async_copy(
                x_hbm.at[idx_v.at[pl.ds(c * ch, ch)]], bufs[c % nbuf], gsem[c % nbuf]
            )

        def start_s(c):
            s_h[c] = pltpu.async_copy(
                bufs[c % nbuf], o_hbm.at[pl.ds(base + c * ch, ch)], ssem[c % nbuf]
            )

        prime = min(nbuf - 1, nchunks)
        for c in range(prime):
            start_g(c)
        for c in range(nchunks):
            g_h[c].wait()
            start_s(c)
            nxt = c + nbuf - 1
            if nxt < nchunks:
                if nxt - nbuf >= 0:
                    s_h[nxt - nbuf].wait()
                start_g(nxt)
        for c in range(max(0, nchunks - nbuf), nchunks):
            s_h[c].wait()

    return gather_kernel(word_table, idx)


def _ln_body(dst_ref, g_ref, p_ref, t_ref, tt_ref, gam_ref, bet_ref, o_ref):
    del dst_ref
    x = g_ref[...] + p_ref[...]
    t0 = t_ref[0, :][None, :]
    dt = (t_ref[1, :] - t_ref[0, :])[None, :]
    x = x + t0 + tt_ref[...] * dt
    h = x.shape[1]
    xb = x.astype(jnp.bfloat16)
    ones = jnp.ones((h, 128), jnp.bfloat16)
    dims = (((1,), (0,)), ((), ()))
    s1 = lax.dot_general(xb, ones, dims, preferred_element_type=jnp.float32)[:, :1]
    s2 = lax.dot_general(xb * xb, ones, dims,
                         preferred_element_type=jnp.float32)[:, :1]
    mean = s1 / h
    var = s2 / h - mean * mean
    y = (x - mean) * lax.rsqrt(var + _EPS)
    o_ref[...] = y * gam_ref[...] + bet_ref[...]


_TC_BLOCK = 1024


def _tc_add_ln_chunk(dst, g_k, pos_table, tt_k, type_table, gamma2, beta2,
                     s0, seq, batch):
    s, h = g_k.shape
    ck = s // batch
    t = _TC_BLOCK
    sub = ck // t
    n_total = batch * seq
    sb_total = seq // t
    p0 = s0 // t
    first = dst is None

    def body(*refs):
        if first:
            _ln_body(None, *refs)
        else:
            _ln_body(*refs)

    specs = [
        pl.BlockSpec((t, h), lambda j, i, _s=sub: (i * _s + j, 0)),
        pl.BlockSpec((t, h), lambda j, i, _p=p0: (_p + j, 0)),
        pl.BlockSpec((2, h), lambda j, i: (0, 0)),
        pl.BlockSpec((t, 1), lambda j, i, _s=sub: (i * _s + j, 0)),
        pl.BlockSpec((1, h), lambda j, i: (0, 0)),
        pl.BlockSpec((1, h), lambda j, i: (0, 0)),
    ]
    args = [g_k, pos_table, type_table, tt_k, gamma2, beta2]
    aliases = {}
    if not first:
        specs = [pl.BlockSpec(memory_space=pl.ANY)] + specs
        args = [dst] + args
        aliases = {0: 0}

    return pl.pallas_call(
        body,
        grid=(sub, batch),
        in_specs=specs,
        out_specs=pl.BlockSpec(
            (t, h),
            lambda j, i, _p=p0, _sb=sb_total: (i * _sb + _p + j, 0)),
        out_shape=jax.ShapeDtypeStruct((n_total, h), jnp.float32),
        input_output_aliases=aliases,
    )(*args)


_CHUNKS = ((0, 1024), (1024, 1024))


def kernel(input_ids, token_type_ids, word_table, pos_table, type_table, gamma, beta):
    batch, seq = input_ids.shape
    h = word_table.shape[1]
    gamma2 = gamma.reshape(1, h)
    beta2 = beta.reshape(1, h)

    gathered = []
    tts = []
    for s0, ck in _CHUNKS:
        ids_k = input_ids[:, s0:s0 + ck].reshape(-1).astype(jnp.int32)
        tts.append(token_type_ids[:, s0:s0 + ck]
                   .reshape(-1, 1).astype(jnp.float32))
        gathered.append(_sc_gather(word_table, ids_k))

    out = None
    for k, (s0, ck) in enumerate(_CHUNKS):
        out = _tc_add_ln_chunk(out, gathered[k], pos_table, tts[k], type_table,
                               gamma2, beta2, s0, seq, batch)
    return out.reshape(batch, seq, h)


# speedup vs baseline: 1.0134x; 1.0134x over previous
import functools

import jax
import jax.numpy as jnp
from jax import lax
from jax.experimental import pallas as pl
from jax.experimental.pallas import tpu as pltpu
from jax.experimental.pallas import tpu_sc as plsc

_EPS = 1e-5
_GATHER_CHUNK = 8
_NBUF = 6


def _sc_gather(word_table, idx):
    n = idx.shape[0]
    h = word_table.shape[1]
    mesh = plsc.VectorSubcoreMesh(core_axis_name="core", subcore_axis_name="subcore")
    num_workers = mesh.num_cores * mesh.num_subcores
    b_per_w = n // num_workers
    ch = _GATHER_CHUNK
    nbuf = _NBUF
    nchunks = b_per_w // ch

    @functools.partial(
        pl.kernel,
        out_type=jax.ShapeDtypeStruct((n, h), jnp.float32),
        mesh=mesh,
        scratch_types=[pltpu.VMEM((b_per_w,), jnp.int32)]
        + [pltpu.VMEM((ch, h), jnp.float32) for _ in range(nbuf)]
        + [pltpu.SemaphoreType.DMA for _ in range(2 * nbuf)],
    )
    def gather_kernel(x_hbm, i_hbm, o_hbm, idx_v, *scratch):
        bufs = scratch[:nbuf]
        gsem = scratch[nbuf:2 * nbuf]
        ssem = scratch[2 * nbuf:]
        wid = lax.axis_index("subcore") * mesh.num_cores + lax.axis_index("core")
        base = wid * b_per_w
        pltpu.sync_copy(i_hbm.at[pl.ds(base, b_per_w)], idx_v)

        g_h = [None] * nchunks
        s_h = [None] * nchunks

        def start_g(c):
            g_h[c] = pltpu.async_copy(
                x_hbm.at[idx_v.at[pl.ds(c * ch, ch)]], bufs[c % nbuf], gsem[c % nbuf]
            )

        def start_s(c):
            s_h[c] = pltpu.async_copy(
                bufs[c % nbuf], o_hbm.at[pl.ds(base + c * ch, ch)], ssem[c % nbuf]
            )

        prime = min(nbuf - 1, nchunks)
        for c in range(prime):
            start_g(c)
        for c in range(nchunks):
            g_h[c].wait()
            start_s(c)
            nxt = c + nbuf - 1
            if nxt < nchunks:
                if nxt - nbuf >= 0:
                    s_h[nxt - nbuf].wait()
                start_g(nxt)
        for c in range(max(0, nchunks - nbuf), nchunks):
            s_h[c].wait()

    return gather_kernel(word_table, idx)


def _ln_body(dst_ref, g_ref, p_ref, t_ref, tt_ref, gam_ref, bet_ref, o_ref):
    del dst_ref
    x = g_ref[...] + p_ref[...]
    t0 = t_ref[0, :][None, :]
    dt = (t_ref[1, :] - t_ref[0, :])[None, :]
    x = x + t0 + tt_ref[...] * dt
    h = x.shape[1]
    xb = x.astype(jnp.bfloat16)
    ones = jnp.ones((h, 128), jnp.bfloat16)
    dims = (((1,), (0,)), ((), ()))
    s1 = lax.dot_general(xb, ones, dims, preferred_element_type=jnp.float32)[:, :1]
    s2 = lax.dot_general(xb * xb, ones, dims,
                         preferred_element_type=jnp.float32)[:, :1]
    mean = s1 / h
    var = s2 / h - mean * mean
    y = (x - mean) * lax.rsqrt(var + _EPS)
    o_ref[...] = y * gam_ref[...] + bet_ref[...]


_TC_BLOCK = 1024


def _tc_add_ln_chunk(dst, g_k, pos_table, tt_k, type_table, gamma2, beta2,
                     s0, seq, batch):
    s, h = g_k.shape
    ck = s // batch
    t = _TC_BLOCK
    sub = ck // t
    n_total = batch * seq
    sb_total = seq // t
    p0 = s0 // t
    first = dst is None

    def body(*refs):
        if first:
            _ln_body(None, *refs)
        else:
            _ln_body(*refs)

    specs = [
        pl.BlockSpec((t, h), lambda j, i, _s=sub: (i * _s + j, 0)),
        pl.BlockSpec((t, h), lambda j, i, _p=p0: (_p + j, 0)),
        pl.BlockSpec((2, h), lambda j, i: (0, 0)),
        pl.BlockSpec((t, 1), lambda j, i, _s=sub: (i * _s + j, 0)),
        pl.BlockSpec((1, h), lambda j, i: (0, 0)),
        pl.BlockSpec((1, h), lambda j, i: (0, 0)),
    ]
    args = [g_k, pos_table, type_table, tt_k, gamma2, beta2]
    aliases = {}
    if not first:
        specs = [pl.BlockSpec(memory_space=pl.ANY)] + specs
        args = [dst] + args
        aliases = {0: 0}

    return pl.pallas_call(
        body,
        grid=(sub, batch),
        in_specs=specs,
        out_specs=pl.BlockSpec(
            (t, h),
            lambda j, i, _p=p0, _sb=sb_total: (i * _sb + _p + j, 0)),
        out_shape=jax.ShapeDtypeStruct((n_total, h), jnp.float32),
        input_output_aliases=aliases,
    )(*args)


_CHUNKS = ((0, 1024), (1024, 1024))


def kernel(input_ids, token_type_ids, word_table, pos_table, type_table, gamma, beta):
    batch, seq = input_ids.shape
    h = word_table.shape[1]
    gamma2 = gamma.reshape(1, h)
    beta2 = beta.reshape(1, h)

    gathered = []
    tts = []
    for s0, ck in _CHUNKS:
        ids_k = input_ids[:, s0:s0 + ck].reshape(-1).astype(jnp.int32)
        tts.append(token_type_ids[:, s0:s0 + ck]
                   .reshape(-1, 1).astype(jnp.float32))
        gathered.append(_sc_gather(word_table, ids_k))

    out = None
    for k, (s0, ck) in enumerate(_CHUNKS):
        out = _tc_add_ln_chunk(out, gathered[k], pos_table, tts[k], type_table,
                               gamma2, beta2, s0, seq, batch)
    return out.reshape(batch, seq, h)
